# baseline (speedup 1.0000x reference)
import functools

import jax
import jax.numpy as jnp
from jax import lax
from jax.experimental import pallas as pl
from jax.experimental.pallas import tpu as pltpu
from jax.experimental.pallas import tpu_sc as plsc

N_NODES = 10000
N_EDGES = 160000
D_IN = 512
D_OUT = 256

NC = 2
NS = 16
CHUNK = 125
CHUNKS_PER_TILE = 80
DH = D_OUT // NC
STRIPE = 632
ACC_ROWS = NS * STRIPE
ROW_BLK = 2000


def _matmul_body(x_ref, w_ref, o_ref):
    o_ref[0, :, :] = lax.dot_general(
        x_ref[...].astype(jnp.bfloat16), w_ref[...].astype(jnp.bfloat16),
        (((1,), (0,)), ((), ())),
        preferred_element_type=jnp.float32)


def _support_split(x, W):
    return pl.pallas_call(
        _matmul_body,
        grid=(NC, N_NODES // ROW_BLK),
        in_specs=[
            pl.BlockSpec((ROW_BLK, D_IN), lambda c, i: (i, 0)),
            pl.BlockSpec((D_IN, DH), lambda c, i: (0, c)),
        ],
        out_specs=pl.BlockSpec((1, ROW_BLK, DH), lambda c, i: (c, i, 0)),
        out_shape=jax.ShapeDtypeStruct((NC, N_NODES, DH), jnp.float32),
    )(x, W)


@functools.partial(
    pl.kernel,
    mesh=plsc.VectorSubcoreMesh(core_axis_name="c", subcore_axis_name="s"),
    out_type=jax.ShapeDtypeStruct((N_NODES, D_OUT), jnp.float32),
    scratch_types=[
        pltpu.VMEM((CHUNKS_PER_TILE, CHUNK), jnp.int32),
        pltpu.VMEM((CHUNKS_PER_TILE, CHUNK), jnp.int32),
        pltpu.VMEM((CHUNK, DH), jnp.float32),
        pltpu.VMEM_SHARED((ACC_ROWS, DH), jnp.float32),
        pltpu.SemaphoreType.DMA,
    ],
)
def _sc_aggregate(support_hbm, src_hbm, dst_hbm, zeros_hbm, out_hbm,
                  src_v, dst_v, msg_v, acc_sh, sem):
    c = lax.axis_index("c")
    s = lax.axis_index("s")
    tbl = support_hbm.at[c]

    pltpu.sync_copy(zeros_hbm.at[pl.ds(s * STRIPE, STRIPE)],
                    acc_sh.at[pl.ds(s * STRIPE, STRIPE)])
    pltpu.sync_copy(src_hbm.at[s], src_v)
    pltpu.sync_copy(dst_hbm.at[s], dst_v)
    plsc.subcore_barrier()

    @pl.loop(0, CHUNKS_PER_TILE)
    def _(j):
        pltpu.async_copy(tbl.at[src_v.at[j]], msg_v, sem).wait()
        pltpu.sync_copy(msg_v, acc_sh.at[dst_v.at[j]], add=True)

    plsc.subcore_barrier()

    @pl.when(s < NS - 1)
    def _():
        pltpu.sync_copy(acc_sh.at[pl.ds(s * STRIPE, STRIPE)],
                        out_hbm.at[pl.ds(s * STRIPE, STRIPE), pl.ds(c * DH, DH)])

    @pl.when(s == NS - 1)
    def _():
        last = N_NODES - (NS - 1) * STRIPE
        pltpu.sync_copy(acc_sh.at[pl.ds((NS - 1) * STRIPE, last)],
                        out_hbm.at[pl.ds((NS - 1) * STRIPE, last), pl.ds(c * DH, DH)])


def kernel(adj, x, W):
    support = _support_split(x, W)

    src4 = adj[0].reshape(NS, CHUNKS_PER_TILE, CHUNK)
    dst4 = adj[1].reshape(NS, CHUNKS_PER_TILE, CHUNK)
    zeros = jnp.zeros((ACC_ROWS, DH), jnp.float32)

    return _sc_aggregate(support, src4, dst4, zeros)


# speedup vs baseline: 1.0043x; 1.0043x over previous
import functools

import jax
import jax.numpy as jnp
from jax import lax
from jax.experimental import pallas as pl
from jax.experimental.pallas import tpu as pltpu
from jax.experimental.pallas import tpu_sc as plsc

N_NODES = 10000
N_EDGES = 160000
D_IN = 512
D_OUT = 256

NC = 2
NS = 16
CHUNK = 125
CHUNKS_PER_TILE = 80
DH = D_OUT // NC
STRIPE = 632
ACC_ROWS = NS * STRIPE
ROW_BLK = 2000


def _matmul_body(x_ref, w_ref, o_ref):
    o_ref[0, :, :] = lax.dot_general(
        x_ref[...].astype(jnp.bfloat16), w_ref[...].astype(jnp.bfloat16),
        (((1,), (0,)), ((), ())),
        preferred_element_type=jnp.float32)


def _support_split(x, W):
    return pl.pallas_call(
        _matmul_body,
        grid=(NC, N_NODES // ROW_BLK),
        in_specs=[
            pl.BlockSpec((ROW_BLK, D_IN), lambda c, i: (i, 0)),
            pl.BlockSpec((D_IN, DH), lambda c, i: (0, c)),
        ],
        out_specs=pl.BlockSpec((1, ROW_BLK, DH), lambda c, i: (c, i, 0)),
        out_shape=jax.ShapeDtypeStruct((NC, N_NODES, DH), jnp.float32),
    )(x, W)


@functools.partial(
    pl.kernel,
    mesh=plsc.VectorSubcoreMesh(core_axis_name="c", subcore_axis_name="s"),
    out_type=jax.ShapeDtypeStruct((N_NODES, D_OUT), jnp.float32),
    scratch_types=[
        pltpu.VMEM((CHUNKS_PER_TILE, CHUNK), jnp.int32),
        pltpu.VMEM((CHUNKS_PER_TILE, CHUNK), jnp.int32),
        pltpu.VMEM((CHUNK, DH), jnp.float32),
        pltpu.VMEM_SHARED((ACC_ROWS, DH), jnp.float32),
        pltpu.SemaphoreType.DMA,
    ],
)
def _sc_aggregate(support_hbm, src_hbm, dst_hbm, zeros_hbm, out_hbm,
                  src_v, dst_v, msg_v, acc_sh, sem):
    c = lax.axis_index("c")
    s = lax.axis_index("s")
    tbl = support_hbm.at[c]

    pltpu.sync_copy(zeros_hbm, acc_sh.at[pl.ds(s * STRIPE, STRIPE)])
    pltpu.sync_copy(src_hbm.at[s], src_v)
    pltpu.sync_copy(dst_hbm.at[s], dst_v)
    plsc.subcore_barrier()

    @pl.loop(0, CHUNKS_PER_TILE)
    def _(j):
        pltpu.async_copy(tbl.at[src_v.at[j]], msg_v, sem).wait()
        pltpu.sync_copy(msg_v, acc_sh.at[dst_v.at[j]], add=True)

    plsc.subcore_barrier()

    @pl.when(s < NS - 1)
    def _():
        pltpu.sync_copy(acc_sh.at[pl.ds(s * STRIPE, STRIPE)],
                        out_hbm.at[pl.ds(s * STRIPE, STRIPE), pl.ds(c * DH, DH)])

    @pl.when(s == NS - 1)
    def _():
        last = N_NODES - (NS - 1) * STRIPE
        pltpu.sync_copy(acc_sh.at[pl.ds((NS - 1) * STRIPE, last)],
                        out_hbm.at[pl.ds((NS - 1) * STRIPE, last), pl.ds(c * DH, DH)])


def kernel(adj, x, W):
    support = _support_split(x, W)

    src4 = adj[0].reshape(NS, CHUNKS_PER_TILE, CHUNK)
    dst4 = adj[1].reshape(NS, CHUNKS_PER_TILE, CHUNK)
    zeros = jnp.zeros((STRIPE, DH), jnp.float32)

    return _sc_aggregate(support, src4, dst4, zeros)
